# baseline (speedup 1.0000x reference)
import jax
import jax.numpy as jnp
from jax import lax
from jax.experimental import pallas as pl
from jax.experimental.pallas import tpu as pltpu
from jax.experimental.pallas import tpu_sc as plsc

N = 256
S = 256
D = 256
B = N * S
NC = 2
NS = 16
NW = NC * NS
ROWS_PER_W = B // NW
JC = 64
STEPS = ROWS_PER_W // JC
NBUF = 3
LANES = 16


def _sc_body(x_hbm, table_hbm, out_hbm, idx_v, x_v, bufs, gsems, ssems,
             xsem):
    wid = lax.axis_index("s") * NC + lax.axis_index("c")
    base = wid * ROWS_PER_W

    pltpu.sync_copy(x_hbm.at[pl.ds(base, ROWS_PER_W)], idx_v)
    x_copy = pltpu.make_async_copy(x_hbm, x_v, xsem)
    x_copy.start()

    def gather(s):
        b = s % NBUF
        return pltpu.make_async_copy(
            table_hbm.at[idx_v.at[pl.ds(JC * s, JC)]], bufs[b], gsems[b])

    def store(s):
        b = s % NBUF
        return pltpu.make_async_copy(
            bufs[b], out_hbm.at[pl.ds(base + JC * s, JC)], ssems[b])

    def add_chunk(s):
        buf = bufs[s % NBUF]
        j0 = JC * (s % (S // JC))

        def row_body(r, _):
            a = [x_v[pl.ds((j0 + r) * D + c * LANES, LANES)]
                 .astype(jnp.float32) for c in range(D // LANES)]
            for c in range(D // LANES):
                plsc.addupdate(buf.at[r, pl.ds(c * LANES, LANES)], a[c])
            return 0

        lax.fori_loop(0, JC, row_body, 0)

    for s in range(NBUF):
        gather(s).start()
    x_copy.wait()
    for s in range(STEPS):
        gather(s).wait()
        add_chunk(s)
        store(s).start()
        if s + NBUF < STEPS:
            store(s).wait()
            gather(s + NBUF).start()
    for s in range(STEPS - NBUF, STEPS):
        store(s).wait()


@jax.jit
def _pe_lookup(x_flat, table):
    mesh = plsc.VectorSubcoreMesh(core_axis_name="c", subcore_axis_name="s")
    return pl.kernel(
        _sc_body,
        out_type=jax.ShapeDtypeStruct((B, D), jnp.float32),
        mesh=mesh,
        scratch_types=[
            pltpu.VMEM((ROWS_PER_W,), jnp.int32),
            pltpu.VMEM((B,), jnp.int32),
            tuple(pltpu.VMEM((JC, D), jnp.float32) for _ in range(NBUF)),
            tuple(pltpu.SemaphoreType.DMA for _ in range(NBUF)),
            tuple(pltpu.SemaphoreType.DMA for _ in range(NBUF)),
            pltpu.SemaphoreType.DMA,
        ],
    )(x_flat, table)


def kernel(x, encoding_weight):
    out = _pe_lookup(x.reshape(-1), encoding_weight)
    return out.reshape(N, S, D)
